# Initial kernel scaffold; baseline (speedup 1.0000x reference)
import jax
import jax.numpy as jnp
from jax.experimental import pallas as pl


def kernel(x, table, W1, b1, W2, b2, W3, b3):
    raise NotImplementedError("write your pallas kernel here")


# speedup vs baseline: 7.6180x; 7.6180x over previous
import functools

import jax
import jax.numpy as jnp
from jax import lax
from jax.experimental import pallas as pl
from jax.experimental.pallas import tpu as pltpu
from jax.experimental.pallas import tpu_sc as plsc

_VOCAB = 100000
_EMB = 128
_HID = 64
_OUT = 3
_B = 4096
_L = 200
_NCHUNK = 2
_CHUNK = _L // _NCHUNK
_LANES = 16
_NVREG = _EMB // _LANES


@functools.lru_cache(maxsize=None)
def _make_pool_kernel():
    info = plsc.get_sparse_core_info()
    nc, ns = info.num_cores, info.num_subcores
    nw = nc * ns
    bpw = _B // nw

    mesh = plsc.VectorSubcoreMesh(core_axis_name="c", subcore_axis_name="s")

    @functools.partial(
        pl.kernel,
        mesh=mesh,
        out_type=jax.ShapeDtypeStruct((_B, _EMB), jnp.float32),
        scratch_types=[
            pltpu.VMEM((bpw, _NCHUNK, _CHUNK), jnp.int32),
            pltpu.VMEM((_L, _EMB), jnp.float32),
            pltpu.VMEM((bpw, _EMB), jnp.float32),
            pltpu.SemaphoreType.DMA,
        ],
    )
    def pool(x_hbm, table_hbm, out_hbm, idx_v, rows_v, acc_v, sem):
        wid = lax.axis_index("s") * nc + lax.axis_index("c")
        base = wid * bpw
        pltpu.sync_copy(x_hbm.at[pl.ds(base, bpw)], idx_v)

        def sample_body(s, carry):
            cp0 = pltpu.async_copy(
                table_hbm.at[idx_v.at[s, 0]], rows_v.at[pl.ds(0, _CHUNK)], sem
            )
            cp1 = pltpu.async_copy(
                table_hbm.at[idx_v.at[s, 1]], rows_v.at[pl.ds(_CHUNK, _CHUNK)], sem
            )
            cp0.wait()
            cp1.wait()

            def row_body(r, acc):
                return tuple(
                    acc[c] + rows_v[r, pl.ds(_LANES * c, _LANES)]
                    for c in range(_NVREG)
                )

            acc = lax.fori_loop(
                0, _L, row_body,
                tuple(jnp.zeros((_LANES,), jnp.float32) for _ in range(_NVREG)),
            )
            for c in range(_NVREG):
                acc_v[s, pl.ds(_LANES * c, _LANES)] = acc[c]
            return carry

        lax.fori_loop(0, bpw, sample_body, 0)
        pltpu.sync_copy(acc_v, out_hbm.at[pl.ds(base, bpw)])

    return pool


def _mlp_body(p_ref, w1_ref, b1_ref, w2_ref, b2_ref, w3_ref, b3_ref, o_ref):
    h = p_ref[...] * (1.0 / _L)
    h = jnp.maximum(
        jnp.dot(h, w1_ref[...], preferred_element_type=jnp.float32) + b1_ref[...],
        0.0,
    )
    h = jnp.maximum(
        jnp.dot(h, w2_ref[...], preferred_element_type=jnp.float32) + b2_ref[...],
        0.0,
    )
    o_ref[...] = (
        jnp.dot(h, w3_ref[...], preferred_element_type=jnp.float32) + b3_ref[...]
    )


def kernel(x, table, W1, b1, W2, b2, W3, b3):
    x3 = x.astype(jnp.int32).reshape(_B, _NCHUNK, _CHUNK)
    pooled = _make_pool_kernel()(x3, table)
    out = pl.pallas_call(
        _mlp_body,
        out_shape=jax.ShapeDtypeStruct((_B, _OUT), jnp.float32),
    )(
        pooled,
        W1,
        b1.reshape(1, -1),
        W2,
        b2.reshape(1, -1),
        W3,
        b3.reshape(1, -1),
    )
    return out


# speedup vs baseline: 13.1919x; 1.7317x over previous
import functools

import jax
import jax.numpy as jnp
from jax import lax
from jax.experimental import pallas as pl
from jax.experimental.pallas import tpu as pltpu
from jax.experimental.pallas import tpu_sc as plsc

_VOCAB = 100000
_EMB = 128
_HID = 64
_OUT = 3
_B = 4096
_L = 200
_NCHUNK = 2
_CHUNK = _L // _NCHUNK
_LANES = 16
_NVREG = _EMB // _LANES
_UNROLL = 8


@functools.lru_cache(maxsize=None)
def _make_pool_kernel():
    info = plsc.get_sparse_core_info()
    nc, ns = info.num_cores, info.num_subcores
    nw = nc * ns
    bpw = _B // nw

    mesh = plsc.VectorSubcoreMesh(core_axis_name="c", subcore_axis_name="s")

    @functools.partial(
        pl.kernel,
        mesh=mesh,
        out_type=jax.ShapeDtypeStruct((_B, _EMB), jnp.float32),
        scratch_types=[
            pltpu.VMEM((bpw, _NCHUNK, _CHUNK), jnp.int32),
            pltpu.VMEM((2, _L, _EMB), jnp.float32),
            pltpu.VMEM((bpw, _EMB), jnp.float32),
            pltpu.SemaphoreType.DMA,
            pltpu.SemaphoreType.DMA,
        ],
    )
    def pool(x_hbm, table_hbm, out_hbm, idx_v, rows_v, acc_v, sem0, sem1):
        wid = lax.axis_index("s") * nc + lax.axis_index("c")
        base = wid * bpw
        pltpu.sync_copy(x_hbm.at[pl.ds(base, bpw)], idx_v)
        sems = (sem0, sem1)

        def issue(s, buf):
            pltpu.async_copy(
                table_hbm.at[idx_v.at[s, 0]],
                rows_v.at[buf, pl.ds(0, _CHUNK)],
                sems[buf],
            )
            pltpu.async_copy(
                table_hbm.at[idx_v.at[s, 1]],
                rows_v.at[buf, pl.ds(_CHUNK, _CHUNK)],
                sems[buf],
            )

        def wait_buf(buf):
            pltpu.make_async_copy(
                table_hbm.at[pl.ds(0, _L)], rows_v.at[buf], sems[buf]
            ).wait()

        def accumulate(s, buf):
            def row_body(i, acc):
                for u in range(_UNROLL):
                    r = i * _UNROLL + u
                    acc = tuple(
                        acc[c] + rows_v[buf, r, pl.ds(_LANES * c, _LANES)]
                        for c in range(_NVREG)
                    )
                return acc

            acc = lax.fori_loop(
                0, _L // _UNROLL, row_body,
                tuple(jnp.zeros((_LANES,), jnp.float32) for _ in range(_NVREG)),
            )
            for c in range(_NVREG):
                acc_v[s, pl.ds(_LANES * c, _LANES)] = acc[c]

        issue(0, 0)
        issue(1, 1)

        def pair_body(i, carry):
            s0 = 2 * i
            s1 = s0 + 1
            wait_buf(0)
            accumulate(s0, 0)
            issue(jnp.minimum(s0 + 2, bpw - 1), 0)
            wait_buf(1)
            accumulate(s1, 1)
            issue(jnp.minimum(s1 + 2, bpw - 1), 1)
            return carry

        lax.fori_loop(0, bpw // 2, pair_body, 0)
        wait_buf(0)
        wait_buf(1)
        pltpu.sync_copy(acc_v, out_hbm.at[pl.ds(base, bpw)])

    return pool


def _mlp_body(p_ref, w1_ref, b1_ref, w2_ref, b2_ref, w3_ref, b3_ref, o_ref):
    h = p_ref[...] * (1.0 / _L)
    h = jnp.maximum(
        jnp.dot(h, w1_ref[...], preferred_element_type=jnp.float32) + b1_ref[...],
        0.0,
    )
    h = jnp.maximum(
        jnp.dot(h, w2_ref[...], preferred_element_type=jnp.float32) + b2_ref[...],
        0.0,
    )
    o_ref[...] = (
        jnp.dot(h, w3_ref[...], preferred_element_type=jnp.float32) + b3_ref[...]
    )


def kernel(x, table, W1, b1, W2, b2, W3, b3):
    x3 = x.astype(jnp.int32).reshape(_B, _NCHUNK, _CHUNK)
    pooled = _make_pool_kernel()(x3, table)
    out = pl.pallas_call(
        _mlp_body,
        out_shape=jax.ShapeDtypeStruct((_B, _OUT), jnp.float32),
    )(
        pooled,
        W1,
        b1.reshape(1, -1),
        W2,
        b2.reshape(1, -1),
        W3,
        b3.reshape(1, -1),
    )
    return out


# speedup vs baseline: 16.1811x; 1.2266x over previous
import functools

import jax
import jax.numpy as jnp
from jax import lax
from jax.experimental import pallas as pl
from jax.experimental.pallas import tpu as pltpu
from jax.experimental.pallas import tpu_sc as plsc

_VOCAB = 100000
_EMB = 128
_HID = 64
_OUT = 3
_B = 4096
_L = 200
_NCHUNK = 2
_CHUNK = _L // _NCHUNK
_NBUF = 3
_LANES = 16
_NVREG = _EMB // _LANES
_UNROLL = 8


@functools.lru_cache(maxsize=None)
def _make_pool_kernel():
    info = plsc.get_sparse_core_info()
    nc, ns = info.num_cores, info.num_subcores
    nw = nc * ns
    bpw = _B // nw

    mesh = plsc.VectorSubcoreMesh(core_axis_name="c", subcore_axis_name="s")

    @functools.partial(
        pl.kernel,
        mesh=mesh,
        out_type=jax.ShapeDtypeStruct((_B, _EMB), jnp.float32),
        scratch_types=[
            pltpu.VMEM((bpw, _NCHUNK, _CHUNK), jnp.int32),
            pltpu.VMEM((_NBUF, _L, _EMB), jnp.float32),
            pltpu.VMEM((bpw, _EMB), jnp.float32),
        ] + [pltpu.SemaphoreType.DMA] * _NBUF,
    )
    def pool(x_hbm, table_hbm, out_hbm, idx_v, rows_v, acc_v, *sems):
        wid = lax.axis_index("s") * nc + lax.axis_index("c")
        base = wid * bpw
        pltpu.sync_copy(x_hbm.at[pl.ds(base, bpw)], idx_v)

        def issue(s, buf):
            for j in range(_NCHUNK):
                pltpu.async_copy(
                    table_hbm.at[idx_v.at[s, j]],
                    rows_v.at[buf, pl.ds(j * _CHUNK, _CHUNK)],
                    sems[buf],
                )

        def wait_buf(buf):
            pltpu.make_async_copy(
                table_hbm.at[pl.ds(0, _L)], rows_v.at[buf], sems[buf]
            ).wait()

        def accumulate(s, buf):
            def row_body(i, a):
                for u in range(_UNROLL):
                    r = i * _UNROLL + u
                    a = tuple(
                        a[c] + rows_v[buf, r, pl.ds(_LANES * c, _LANES)]
                        for c in range(_NVREG)
                    )
                return a

            acc = lax.fori_loop(
                0, _L // _UNROLL, row_body,
                tuple(jnp.zeros((_LANES,), jnp.float32) for _ in range(_NVREG)),
            )
            for c in range(_NVREG):
                acc_v[s, pl.ds(_LANES * c, _LANES)] = acc[c]

        for b in range(_NBUF):
            issue(b, b)

        nsup = (bpw - _NBUF) // _NBUF

        def super_body(i, carry):
            s0 = _NBUF * i
            for k in range(_NBUF):
                s = s0 + k
                wait_buf(k)
                accumulate(s, k)
                issue(s + _NBUF, k)
            return carry

        lax.fori_loop(0, nsup, super_body, 0)

        rem = bpw - nsup * _NBUF
        for t in range(rem):
            s = bpw - rem + t
            buf = t % _NBUF
            wait_buf(buf)
            accumulate(s, buf)
            if s + _NBUF < bpw:
                issue(s + _NBUF, buf)

        pltpu.sync_copy(acc_v, out_hbm.at[pl.ds(base, bpw)])

    return pool


def _mlp_body(p_ref, w1_ref, b1_ref, w2_ref, b2_ref, w3_ref, b3_ref, o_ref):
    h = p_ref[...] * (1.0 / _L)
    h = jnp.maximum(
        jnp.dot(h, w1_ref[...], preferred_element_type=jnp.float32) + b1_ref[...],
        0.0,
    )
    h = jnp.maximum(
        jnp.dot(h, w2_ref[...], preferred_element_type=jnp.float32) + b2_ref[...],
        0.0,
    )
    o_ref[...] = (
        jnp.dot(h, w3_ref[...], preferred_element_type=jnp.float32) + b3_ref[...]
    )


def kernel(x, table, W1, b1, W2, b2, W3, b3):
    x3 = x.astype(jnp.int32).reshape(_B, _NCHUNK, _CHUNK)
    pooled = _make_pool_kernel()(x3, table)
    out = pl.pallas_call(
        _mlp_body,
        out_shape=jax.ShapeDtypeStruct((_B, _OUT), jnp.float32),
    )(
        pooled,
        W1,
        b1.reshape(1, -1),
        W2,
        b2.reshape(1, -1),
        W3,
        b3.reshape(1, -1),
    )
    return out
